# baseline (speedup 1.0000x reference)
import jax
import jax.numpy as jnp
from jax import lax
from jax.experimental import pallas as pl
from jax.experimental.pallas import tpu as pltpu
from jax.experimental.pallas import tpu_sc as plsc

N = 10000
D = 128
E = 320000
L = 100000

NC, NS = 2, 16
NW = NC * NS
C = 128
NCHUNK = 80
EPT = NCHUNK * C
NPAD = 10240
RPS = NPAD // NS
LCHUNK = 26
LPT = LCHUNK * C
LREAL = L // NW

_mesh = plsc.VectorSubcoreMesh(core_axis_name="c", subcore_axis_name="s")


def _pad_tiles(arr, per_tile, pad_per_tile, pad_vals):
    a = arr.reshape(NW, per_tile)
    pad = jnp.broadcast_to(pad_vals[None, :], (NW, pad_per_tile))
    return jnp.concatenate([a, pad], axis=1).reshape(-1)


def _zero16():
    return jnp.zeros((16,), jnp.float32)


def _agg2_body(with_count, xtA, xtB, srcA, dstA, srcB, dstB, *rest):
    if with_count:
        (outA, outB, cntA, cntB,
         sixA, dixA, rows, zbuf, ones, zcnt, accS, cntS, gsem) = rest
    else:
        (outA, outB,
         sixA, dixA, rows, zbuf, ones, zcnt, accS, cntS, gsem) = rest
        cntA = cntB = None
    cid = lax.axis_index("c")
    sid = lax.axis_index("s")
    wid = cid * NS + sid

    z16 = _zero16()
    for i in range(64):
        for j in range(D // 16):
            zbuf[i, pl.ds(j * 16, 16)] = z16
    if with_count:
        one16 = jnp.ones((16,), jnp.float32)
        for j in range(C // 16):
            ones[pl.ds(j * 16, 16)] = one16
        for j in range(RPS // 16):
            zcnt[pl.ds(j * 16, 16)] = z16

    def one_conv(xt, srcF, dstF, sidx, didx, out, cnt_out):
        ebase = wid * EPT
        for t in range(RPS // 64):
            pltpu.sync_copy(zbuf, accS.at[pl.ds(sid * RPS + t * 64, 64)])
        if with_count:
            pltpu.sync_copy(zcnt, cntS.at[pl.ds(sid * RPS, RPS)])
        plsc.subcore_barrier()

        def _chunk(k, _):
            b = pl.multiple_of(ebase + k * C, 8)
            pltpu.sync_copy(srcF.at[pl.ds(b, C)], sidx)
            pltpu.sync_copy(dstF.at[pl.ds(b, C)], didx)
            pltpu.async_copy(xt.at[sidx], rows, gsem).wait()
            pltpu.sync_copy(rows, accS.at[didx], add=True)
            if with_count:
                pltpu.sync_copy(ones, cntS.at[didx], add=True)
            return 0

        lax.fori_loop(0, NCHUNK, _chunk, 0)

        plsc.subcore_barrier()
        rr = sid * RPS
        pltpu.sync_copy(accS.at[pl.ds(rr, RPS)], out.at[cid, pl.ds(rr, RPS)])
        if with_count:
            pltpu.sync_copy(cntS.at[pl.ds(rr, RPS)],
                            cnt_out.at[cid, pl.ds(rr, RPS)])
        plsc.subcore_barrier()

    one_conv(xtA, srcA, dstA, sixA, dixA, outA, cntA)
    one_conv(xtB, srcB, dstB, sixA, dixA, outB, cntB)


import functools as _ft


def _make_agg2(with_count):
    outs = [jax.ShapeDtypeStruct((NC, NPAD, D), jnp.float32),
            jax.ShapeDtypeStruct((NC, NPAD, D), jnp.float32)]
    if with_count:
        outs += [jax.ShapeDtypeStruct((NC, NPAD), jnp.float32),
                 jax.ShapeDtypeStruct((NC, NPAD), jnp.float32)]
    return pl.kernel(
        _ft.partial(_agg2_body, with_count),
        out_type=tuple(outs),
        mesh=_mesh,
        scratch_types=[
            pltpu.VMEM((C,), jnp.int32),
            pltpu.VMEM((C,), jnp.int32),
            pltpu.VMEM((C, D), jnp.float32),
            pltpu.VMEM((64, D), jnp.float32),
            pltpu.VMEM((C,), jnp.float32),
            pltpu.VMEM((RPS,), jnp.float32),
            pltpu.VMEM_SHARED((NPAD, D), jnp.float32),
            pltpu.VMEM_SHARED((NPAD,), jnp.float32),
            pltpu.SemaphoreType.DMA,
        ],
    )


_agg2_cnt = _make_agg2(True)
_agg2 = _make_agg2(False)


def _dec_body(zu, zm, srcA, dstA, out, sidx, didx, rs, rd, obuf, sem):
    cid = lax.axis_index("c")
    sid = lax.axis_index("s")
    wid = cid * NS + sid
    base = wid * LPT
    pltpu.sync_copy(srcA.at[wid], sidx)
    pltpu.sync_copy(dstA.at[wid], didx)
    lane = lax.iota(jnp.int32, 16)

    def _chunk(k, _):
        pltpu.async_copy(zu.at[sidx.at[k]], rs, sem).wait()
        pltpu.async_copy(zm.at[didx.at[k]], rd, sem).wait()

        def grp(g, _):
            dv = jnp.zeros((16,), jnp.float32)
            for q in range(16):
                p = g * 16 + q
                a = rs[p, pl.ds(0, 16)] * rd[p, pl.ds(0, 16)]
                for j in range(1, D // 16):
                    a = a + rs[p, pl.ds(j * 16, 16)] * rd[p, pl.ds(j * 16, 16)]
                for sh in (8, 4, 2, 1):
                    a = a + a[lane ^ sh]
                dv = jnp.where(lane == q, a, dv)
            obuf[pl.ds(g * 16, 16)] = dv
            return 0

        lax.fori_loop(0, C // 16, grp, 0)
        pltpu.sync_copy(obuf, out.at[pl.ds(base + k * C, C)])
        return 0

    lax.fori_loop(0, LCHUNK, _chunk, 0)


_decoder = pl.kernel(
    _dec_body,
    out_type=jax.ShapeDtypeStruct((NW * LPT,), jnp.float32),
    mesh=_mesh,
    scratch_types=[
        pltpu.VMEM((LCHUNK, C), jnp.int32),
        pltpu.VMEM((LCHUNK, C), jnp.int32),
        pltpu.VMEM((C, D), jnp.float32),
        pltpu.VMEM((C, D), jnp.float32),
        pltpu.VMEM((C,), jnp.float32),
        pltpu.SemaphoreType.DMA,
    ],
)


_RB = 1024
_GRID = NPAD // _RB


def _mm_body(x, w, o):
    o[...] = jnp.dot(x[...], w[...], preferred_element_type=jnp.float32)


def _mm(x, w):
    return pl.pallas_call(
        _mm_body,
        grid=(_GRID,),
        in_specs=[
            pl.BlockSpec((_RB, D), lambda i: (i, 0)),
            pl.BlockSpec((D, D), lambda i: (0, 0)),
        ],
        out_specs=pl.BlockSpec((_RB, D), lambda i: (i, 0)),
        out_shape=jax.ShapeDtypeStruct((NPAD, D), jnp.float32),
    )(x, w)


def _fin1_body(pacc, pcnt, x, wroot, wnext, b, h, t):
    acc = pacc[0] + pacc[1]
    cnt = pcnt[0] + pcnt[1]
    inv = 1.0 / jnp.maximum(cnt, 1.0)
    hv = jnp.maximum(
        acc * inv[:, None]
        + jnp.dot(x[...], wroot[...], preferred_element_type=jnp.float32)
        + b[...], 0.0)
    h[...] = hv
    t[...] = jnp.dot(hv, wnext[...], preferred_element_type=jnp.float32)


def _finish1(pacc, pcnt, x, wroot, b, wnext):
    return pl.pallas_call(
        _fin1_body,
        grid=(_GRID,),
        in_specs=[
            pl.BlockSpec((NC, _RB, D), lambda i: (0, i, 0)),
            pl.BlockSpec((NC, _RB), lambda i: (0, i)),
            pl.BlockSpec((_RB, D), lambda i: (i, 0)),
            pl.BlockSpec((D, D), lambda i: (0, 0)),
            pl.BlockSpec((D, D), lambda i: (0, 0)),
            pl.BlockSpec((1, D), lambda i: (0, 0)),
        ],
        out_specs=[
            pl.BlockSpec((_RB, D), lambda i: (i, 0)),
            pl.BlockSpec((_RB, D), lambda i: (i, 0)),
        ],
        out_shape=[
            jax.ShapeDtypeStruct((NPAD, D), jnp.float32),
            jax.ShapeDtypeStruct((NPAD, D), jnp.float32),
        ],
    )(pacc, pcnt, x, wroot, wnext, b.reshape(1, D))


def _fin2_body(pacc, pcnt, x, wroot, b, z):
    acc = pacc[0] + pacc[1]
    cnt = pcnt[0] + pcnt[1]
    inv = 1.0 / jnp.maximum(cnt, 1.0)
    z[...] = (acc * inv[:, None]
              + jnp.dot(x[...], wroot[...], preferred_element_type=jnp.float32)
              + b[...])


def _finish2(pacc, pcnt, x, wroot, b):
    return pl.pallas_call(
        _fin2_body,
        grid=(_GRID,),
        in_specs=[
            pl.BlockSpec((NC, _RB, D), lambda i: (0, i, 0)),
            pl.BlockSpec((NC, _RB), lambda i: (0, i)),
            pl.BlockSpec((_RB, D), lambda i: (i, 0)),
            pl.BlockSpec((D, D), lambda i: (0, 0)),
            pl.BlockSpec((1, D), lambda i: (0, 0)),
        ],
        out_specs=pl.BlockSpec((_RB, D), lambda i: (i, 0)),
        out_shape=jax.ShapeDtypeStruct((NPAD, D), jnp.float32),
    )(pacc, pcnt, x, wroot, b.reshape(1, D))


def kernel(x_user, x_movie,
           W1_um_r, W1_um_root, b1_m, W1_mu_r, W1_mu_root, b1_u,
           W2_um_r, W2_um_root, b2_m, W2_mu_r, W2_mu_root, b2_u,
           edge_index_user_movie, edge_index_movie_user, edge_label_index):
    eum = edge_index_user_movie.astype(jnp.int32)
    emu = edge_index_movie_user.astype(jnp.int32)
    eli = edge_label_index.astype(jnp.int32)
    xp_user = jnp.pad(x_user, ((0, NPAD - N), (0, 0)))
    xp_movie = jnp.pad(x_movie, ((0, NPAD - N), (0, 0)))

    epad = EPT - E // NW
    pad_src = (jnp.arange(epad, dtype=jnp.int32) % 16)
    pad_dst = N + (jnp.arange(epad, dtype=jnp.int32) % (NPAD - N))
    src_um = _pad_tiles(eum[0], E // NW, epad, pad_src)
    dst_um = _pad_tiles(eum[1], E // NW, epad, pad_dst)
    src_mu = _pad_tiles(emu[0], E // NW, epad, pad_src)
    dst_mu = _pad_tiles(emu[1], E // NW, epad, pad_dst)

    lpad = LPT - LREAL
    pad_l = (jnp.arange(lpad, dtype=jnp.int32) % 16)
    src_l = _pad_tiles(eli[0], LREAL, lpad, pad_l).reshape(NW, LCHUNK, C)
    dst_l = _pad_tiles(eli[1], LREAL, lpad, pad_l).reshape(NW, LCHUNK, C)

    t_u1 = _mm(xp_user, W1_um_r)
    t_m1 = _mm(xp_movie, W1_mu_r)
    pacc_m, pacc_u, pcnt_m, pcnt_u = _agg2_cnt(
        t_u1, t_m1, src_um, dst_um, src_mu, dst_mu)
    h_m, t_m2 = _finish1(pacc_m, pcnt_m, xp_movie, W1_um_root, b1_m, W2_mu_r)
    h_u, t_u2 = _finish1(pacc_u, pcnt_u, xp_user, W1_mu_root, b1_u, W2_um_r)

    pacc_m2, pacc_u2 = _agg2(t_u2, t_m2, src_um, dst_um, src_mu, dst_mu)
    z_m = _finish2(pacc_m2, pcnt_m, h_m, W2_um_root, b2_m)
    z_u = _finish2(pacc_u2, pcnt_u, h_u, W2_mu_root, b2_u)

    out_pad = _decoder(z_u, z_m, src_l, dst_l)
    return out_pad.reshape(NW, LPT)[:, :LREAL].reshape(-1)


# speedup vs baseline: 1.0005x; 1.0005x over previous
import jax
import jax.numpy as jnp
from jax import lax
from jax.experimental import pallas as pl
from jax.experimental.pallas import tpu as pltpu
from jax.experimental.pallas import tpu_sc as plsc

N = 10000
D = 128
E = 320000
L = 100000

NC, NS = 2, 16
NW = NC * NS
C = 128
NCHUNK = 80
EPT = NCHUNK * C
NPAD = 10240
RPS = NPAD // NS
LCHUNK = 26
LPT = LCHUNK * C
LREAL = L // NW

_mesh = plsc.VectorSubcoreMesh(core_axis_name="c", subcore_axis_name="s")


def _pad_tiles(arr, per_tile, pad_per_tile, pad_vals):
    a = arr.reshape(NW, per_tile)
    pad = jnp.broadcast_to(pad_vals[None, :], (NW, pad_per_tile))
    return jnp.concatenate([a, pad], axis=1).reshape(-1)


def _zero16():
    return jnp.zeros((16,), jnp.float32)


def _agg2_body(with_count, xtA, xtB, srcA, dstA, srcB, dstB, *rest):
    if with_count:
        (outA, outB, cntA, cntB,
         sixA, dixA, rows, zbuf, ones, zcnt, accS, cntS, gsem) = rest
    else:
        (outA, outB,
         sixA, dixA, rows, zbuf, ones, zcnt, accS, cntS, gsem) = rest
        cntA = cntB = None
    cid = lax.axis_index("c")
    sid = lax.axis_index("s")
    wid = cid * NS + sid

    z16 = _zero16()
    for i in range(64):
        for j in range(D // 16):
            zbuf[i, pl.ds(j * 16, 16)] = z16
    if with_count:
        one16 = jnp.ones((16,), jnp.float32)
        for j in range(C // 16):
            ones[pl.ds(j * 16, 16)] = one16
        for j in range(RPS // 16):
            zcnt[pl.ds(j * 16, 16)] = z16

    def one_conv(xt, srcF, dstF, sidx, didx, out, cnt_out):
        ebase = wid * EPT
        for t in range(RPS // 64):
            pltpu.sync_copy(zbuf, accS.at[pl.ds(sid * RPS + t * 64, 64)])
        if with_count:
            pltpu.sync_copy(zcnt, cntS.at[pl.ds(sid * RPS, RPS)])
        plsc.subcore_barrier()

        @plsc.parallel_loop(0, NCHUNK, 1, unroll=2)
        def _chunk(k):
            b = pl.multiple_of(ebase + k * C, 8)
            pltpu.sync_copy(srcF.at[pl.ds(b, C)], sidx)
            pltpu.sync_copy(dstF.at[pl.ds(b, C)], didx)
            pltpu.async_copy(xt.at[sidx], rows, gsem).wait()
            pltpu.sync_copy(rows, accS.at[didx], add=True)
            if with_count:
                pltpu.sync_copy(ones, cntS.at[didx], add=True)

        plsc.subcore_barrier()
        rr = sid * RPS
        pltpu.sync_copy(accS.at[pl.ds(rr, RPS)], out.at[cid, pl.ds(rr, RPS)])
        if with_count:
            pltpu.sync_copy(cntS.at[pl.ds(rr, RPS)],
                            cnt_out.at[cid, pl.ds(rr, RPS)])
        plsc.subcore_barrier()

    one_conv(xtA, srcA, dstA, sixA, dixA, outA, cntA)
    one_conv(xtB, srcB, dstB, sixA, dixA, outB, cntB)


import functools as _ft


def _make_agg2(with_count):
    outs = [jax.ShapeDtypeStruct((NC, NPAD, D), jnp.float32),
            jax.ShapeDtypeStruct((NC, NPAD, D), jnp.float32)]
    if with_count:
        outs += [jax.ShapeDtypeStruct((NC, NPAD), jnp.float32),
                 jax.ShapeDtypeStruct((NC, NPAD), jnp.float32)]
    return pl.kernel(
        _ft.partial(_agg2_body, with_count),
        out_type=tuple(outs),
        mesh=_mesh,
        scratch_types=[
            pltpu.VMEM((C,), jnp.int32),
            pltpu.VMEM((C,), jnp.int32),
            pltpu.VMEM((C, D), jnp.float32),
            pltpu.VMEM((64, D), jnp.float32),
            pltpu.VMEM((C,), jnp.float32),
            pltpu.VMEM((RPS,), jnp.float32),
            pltpu.VMEM_SHARED((NPAD, D), jnp.float32),
            pltpu.VMEM_SHARED((NPAD,), jnp.float32),
            pltpu.SemaphoreType.DMA,
        ],
    )


_agg2_cnt = _make_agg2(True)
_agg2 = _make_agg2(False)


def _dec_body(zu, zm, srcA, dstA, out, sidx, didx, rs, rd, obuf, sem):
    cid = lax.axis_index("c")
    sid = lax.axis_index("s")
    wid = cid * NS + sid
    base = wid * LPT
    pltpu.sync_copy(srcA.at[wid], sidx)
    pltpu.sync_copy(dstA.at[wid], didx)
    lane = lax.iota(jnp.int32, 16)

    @plsc.parallel_loop(0, LCHUNK, 1, unroll=2)
    def _chunk(k):
        pltpu.async_copy(zu.at[sidx.at[k]], rs, sem).wait()
        pltpu.async_copy(zm.at[didx.at[k]], rd, sem).wait()

        def grp(g, _):
            dv = jnp.zeros((16,), jnp.float32)
            for q in range(16):
                p = g * 16 + q
                a = rs[p, pl.ds(0, 16)] * rd[p, pl.ds(0, 16)]
                for j in range(1, D // 16):
                    a = a + rs[p, pl.ds(j * 16, 16)] * rd[p, pl.ds(j * 16, 16)]
                for sh in (8, 4, 2, 1):
                    a = a + a[lane ^ sh]
                dv = jnp.where(lane == q, a, dv)
            obuf[pl.ds(g * 16, 16)] = dv
            return 0

        lax.fori_loop(0, C // 16, grp, 0)
        pltpu.sync_copy(obuf, out.at[pl.ds(base + k * C, C)])


_decoder = pl.kernel(
    _dec_body,
    out_type=jax.ShapeDtypeStruct((NW * LPT,), jnp.float32),
    mesh=_mesh,
    scratch_types=[
        pltpu.VMEM((LCHUNK, C), jnp.int32),
        pltpu.VMEM((LCHUNK, C), jnp.int32),
        pltpu.VMEM((C, D), jnp.float32),
        pltpu.VMEM((C, D), jnp.float32),
        pltpu.VMEM((C,), jnp.float32),
        pltpu.SemaphoreType.DMA,
    ],
)


_RB = 1024
_GRID = NPAD // _RB


def _mm_body(x, w, o):
    o[...] = jnp.dot(x[...], w[...], preferred_element_type=jnp.float32)


def _mm(x, w):
    return pl.pallas_call(
        _mm_body,
        grid=(_GRID,),
        in_specs=[
            pl.BlockSpec((_RB, D), lambda i: (i, 0)),
            pl.BlockSpec((D, D), lambda i: (0, 0)),
        ],
        out_specs=pl.BlockSpec((_RB, D), lambda i: (i, 0)),
        out_shape=jax.ShapeDtypeStruct((NPAD, D), jnp.float32),
    )(x, w)


def _fin1_body(pacc, pcnt, x, wroot, wnext, b, h, t):
    acc = pacc[0] + pacc[1]
    cnt = pcnt[0] + pcnt[1]
    inv = 1.0 / jnp.maximum(cnt, 1.0)
    hv = jnp.maximum(
        acc * inv[:, None]
        + jnp.dot(x[...], wroot[...], preferred_element_type=jnp.float32)
        + b[...], 0.0)
    h[...] = hv
    t[...] = jnp.dot(hv, wnext[...], preferred_element_type=jnp.float32)


def _finish1(pacc, pcnt, x, wroot, b, wnext):
    return pl.pallas_call(
        _fin1_body,
        grid=(_GRID,),
        in_specs=[
            pl.BlockSpec((NC, _RB, D), lambda i: (0, i, 0)),
            pl.BlockSpec((NC, _RB), lambda i: (0, i)),
            pl.BlockSpec((_RB, D), lambda i: (i, 0)),
            pl.BlockSpec((D, D), lambda i: (0, 0)),
            pl.BlockSpec((D, D), lambda i: (0, 0)),
            pl.BlockSpec((1, D), lambda i: (0, 0)),
        ],
        out_specs=[
            pl.BlockSpec((_RB, D), lambda i: (i, 0)),
            pl.BlockSpec((_RB, D), lambda i: (i, 0)),
        ],
        out_shape=[
            jax.ShapeDtypeStruct((NPAD, D), jnp.float32),
            jax.ShapeDtypeStruct((NPAD, D), jnp.float32),
        ],
    )(pacc, pcnt, x, wroot, wnext, b.reshape(1, D))


def _fin2_body(pacc, pcnt, x, wroot, b, z):
    acc = pacc[0] + pacc[1]
    cnt = pcnt[0] + pcnt[1]
    inv = 1.0 / jnp.maximum(cnt, 1.0)
    z[...] = (acc * inv[:, None]
              + jnp.dot(x[...], wroot[...], preferred_element_type=jnp.float32)
              + b[...])


def _finish2(pacc, pcnt, x, wroot, b):
    return pl.pallas_call(
        _fin2_body,
        grid=(_GRID,),
        in_specs=[
            pl.BlockSpec((NC, _RB, D), lambda i: (0, i, 0)),
            pl.BlockSpec((NC, _RB), lambda i: (0, i)),
            pl.BlockSpec((_RB, D), lambda i: (i, 0)),
            pl.BlockSpec((D, D), lambda i: (0, 0)),
            pl.BlockSpec((1, D), lambda i: (0, 0)),
        ],
        out_specs=pl.BlockSpec((_RB, D), lambda i: (i, 0)),
        out_shape=jax.ShapeDtypeStruct((NPAD, D), jnp.float32),
    )(pacc, pcnt, x, wroot, b.reshape(1, D))


def kernel(x_user, x_movie,
           W1_um_r, W1_um_root, b1_m, W1_mu_r, W1_mu_root, b1_u,
           W2_um_r, W2_um_root, b2_m, W2_mu_r, W2_mu_root, b2_u,
           edge_index_user_movie, edge_index_movie_user, edge_label_index):
    eum = edge_index_user_movie.astype(jnp.int32)
    emu = edge_index_movie_user.astype(jnp.int32)
    eli = edge_label_index.astype(jnp.int32)
    xp_user = jnp.pad(x_user, ((0, NPAD - N), (0, 0)))
    xp_movie = jnp.pad(x_movie, ((0, NPAD - N), (0, 0)))

    epad = EPT - E // NW
    pad_src = (jnp.arange(epad, dtype=jnp.int32) % 16)
    pad_dst = N + (jnp.arange(epad, dtype=jnp.int32) % (NPAD - N))
    src_um = _pad_tiles(eum[0], E // NW, epad, pad_src)
    dst_um = _pad_tiles(eum[1], E // NW, epad, pad_dst)
    src_mu = _pad_tiles(emu[0], E // NW, epad, pad_src)
    dst_mu = _pad_tiles(emu[1], E // NW, epad, pad_dst)

    lpad = LPT - LREAL
    pad_l = (jnp.arange(lpad, dtype=jnp.int32) % 16)
    src_l = _pad_tiles(eli[0], LREAL, lpad, pad_l).reshape(NW, LCHUNK, C)
    dst_l = _pad_tiles(eli[1], LREAL, lpad, pad_l).reshape(NW, LCHUNK, C)

    t_u1 = _mm(xp_user, W1_um_r)
    t_m1 = _mm(xp_movie, W1_mu_r)
    pacc_m, pacc_u, pcnt_m, pcnt_u = _agg2_cnt(
        t_u1, t_m1, src_um, dst_um, src_mu, dst_mu)
    h_m, t_m2 = _finish1(pacc_m, pcnt_m, xp_movie, W1_um_root, b1_m, W2_mu_r)
    h_u, t_u2 = _finish1(pacc_u, pcnt_u, xp_user, W1_mu_root, b1_u, W2_um_r)

    pacc_m2, pacc_u2 = _agg2(t_u2, t_m2, src_um, dst_um, src_mu, dst_mu)
    z_m = _finish2(pacc_m2, pcnt_m, h_m, W2_um_root, b2_m)
    z_u = _finish2(pacc_u2, pcnt_u, h_u, W2_mu_root, b2_u)

    out_pad = _decoder(z_u, z_m, src_l, dst_l)
    return out_pad.reshape(NW, LPT)[:, :LREAL].reshape(-1)


# speedup vs baseline: 1.3986x; 1.3979x over previous
import jax
import jax.numpy as jnp
from jax import lax
from jax.experimental import pallas as pl
from jax.experimental.pallas import tpu as pltpu
from jax.experimental.pallas import tpu_sc as plsc

N = 10000
D = 128
E = 320000
L = 100000

NC, NS = 2, 16
NW = NC * NS
C = 128
NCHUNK = 80
EPT = NCHUNK * C
NPAD = 10240
RPS = NPAD // NS
LCHUNK = 26
LPT = LCHUNK * C
LREAL = L // NW

_mesh = plsc.VectorSubcoreMesh(core_axis_name="c", subcore_axis_name="s")


def _pad_tiles(arr, per_tile, pad_per_tile, pad_vals):
    a = arr.reshape(NW, per_tile)
    pad = jnp.broadcast_to(pad_vals[None, :], (NW, pad_per_tile))
    return jnp.concatenate([a, pad], axis=1).reshape(-1)


def _zero16():
    return jnp.zeros((16,), jnp.float32)


def _agg2_body(with_count, xtA, xtB, srcA, dstA, srcB, dstB, *rest):
    if with_count:
        (outA, outB, cntA, cntB, six0, dix0, six1, dix1, rb0, rb1,
         zbuf, ones, zcnt, accS, cntS, g0, g1, s0, s1, c0, c1) = rest
    else:
        (outA, outB, six0, dix0, six1, dix1, rb0, rb1,
         zbuf, ones, zcnt, accS, cntS, g0, g1, s0, s1, c0, c1) = rest
        cntA = cntB = None
    cid = lax.axis_index("c")
    sid = lax.axis_index("s")
    wid = cid * NS + sid

    z16 = _zero16()
    for i in range(64):
        for j in range(D // 16):
            zbuf[i, pl.ds(j * 16, 16)] = z16
    if with_count:
        one16 = jnp.ones((16,), jnp.float32)
        for j in range(C // 16):
            ones[pl.ds(j * 16, 16)] = one16
        for j in range(RPS // 16):
            zcnt[pl.ds(j * 16, 16)] = z16

    def one_conv(xt, srcF, dstF, out, cnt_out):
        ebase = wid * EPT
        for t in range(RPS // 64):
            pltpu.sync_copy(zbuf, accS.at[pl.ds(sid * RPS + t * 64, 64)])
        if with_count:
            pltpu.sync_copy(zcnt, cntS.at[pl.ds(sid * RPS, RPS)])
        plsc.subcore_barrier()

        six = (six0, six1)
        dix = (dix0, dix1)
        rb = (rb0, rb1)
        gs = (g0, g1)
        ss = (s0, s1)
        cs = (c0, c1)

        def idx_copy(k, p):
            b = pl.multiple_of(ebase + k * C, 8)
            pltpu.sync_copy(srcF.at[pl.ds(b, C)], six[p])
            pltpu.sync_copy(dstF.at[pl.ds(b, C)], dix[p])

        def g_start(k, p):
            pltpu.async_copy(xt.at[six[p]], rb[p], gs[p])

        def g_wait(p):
            pltpu.make_async_copy(xt.at[six[p]], rb[p], gs[p]).wait()

        def s_start(k, p):
            pltpu.async_copy(rb[p], accS.at[dix[p]], ss[p], add=True)
            if with_count:
                pltpu.async_copy(ones, cntS.at[dix[p]], cs[p], add=True)

        def s_wait(p):
            pltpu.make_async_copy(rb[p], accS.at[dix[p]], ss[p]).wait()
            if with_count:
                pltpu.make_async_copy(ones, cntS.at[dix[p]], cs[p]).wait()

        idx_copy(0, 0)
        g_start(0, 0)
        idx_copy(1, 1)
        g_start(1, 1)

        def it(i, _):
            k0 = 2 * i
            g_wait(0)
            s_start(k0, 0)
            g_wait(1)
            s_start(k0 + 1, 1)
            s_wait(0)
            idx_copy(k0 + 2, 0)
            g_start(k0 + 2, 0)
            s_wait(1)
            idx_copy(k0 + 3, 1)
            g_start(k0 + 3, 1)
            return 0

        lax.fori_loop(0, NCHUNK // 2 - 1, it, 0)
        g_wait(0)
        s_start(NCHUNK - 2, 0)
        g_wait(1)
        s_start(NCHUNK - 1, 1)
        s_wait(0)
        s_wait(1)

        plsc.subcore_barrier()
        rr = sid * RPS
        pltpu.sync_copy(accS.at[pl.ds(rr, RPS)], out.at[cid, pl.ds(rr, RPS)])
        if with_count:
            pltpu.sync_copy(cntS.at[pl.ds(rr, RPS)],
                            cnt_out.at[cid, pl.ds(rr, RPS)])
        plsc.subcore_barrier()

    one_conv(xtA, srcA, dstA, outA, cntA)
    one_conv(xtB, srcB, dstB, outB, cntB)


import functools as _ft


def _make_agg2(with_count):
    outs = [jax.ShapeDtypeStruct((NC, NPAD, D), jnp.float32),
            jax.ShapeDtypeStruct((NC, NPAD, D), jnp.float32)]
    if with_count:
        outs += [jax.ShapeDtypeStruct((NC, NPAD), jnp.float32),
                 jax.ShapeDtypeStruct((NC, NPAD), jnp.float32)]
    return pl.kernel(
        _ft.partial(_agg2_body, with_count),
        out_type=tuple(outs),
        mesh=_mesh,
        scratch_types=[
            pltpu.VMEM((C,), jnp.int32),
            pltpu.VMEM((C,), jnp.int32),
            pltpu.VMEM((C,), jnp.int32),
            pltpu.VMEM((C,), jnp.int32),
            pltpu.VMEM((C, D), jnp.float32),
            pltpu.VMEM((C, D), jnp.float32),
            pltpu.VMEM((64, D), jnp.float32),
            pltpu.VMEM((C,), jnp.float32),
            pltpu.VMEM((RPS,), jnp.float32),
            pltpu.VMEM_SHARED((NPAD, D), jnp.float32),
            pltpu.VMEM_SHARED((NPAD,), jnp.float32),
            pltpu.SemaphoreType.DMA,
            pltpu.SemaphoreType.DMA,
            pltpu.SemaphoreType.DMA,
            pltpu.SemaphoreType.DMA,
            pltpu.SemaphoreType.DMA,
            pltpu.SemaphoreType.DMA,
        ],
    )


_agg2_cnt = _make_agg2(True)
_agg2 = _make_agg2(False)


def _dec_body(zu, zm, srcA, dstA, out, sidx, didx, rs, rd, obuf, sem):
    cid = lax.axis_index("c")
    sid = lax.axis_index("s")
    wid = cid * NS + sid
    base = wid * LPT
    pltpu.sync_copy(srcA.at[wid], sidx)
    pltpu.sync_copy(dstA.at[wid], didx)
    lane = lax.iota(jnp.int32, 16)

    def _chunk(k, _):
        pltpu.async_copy(zu.at[sidx.at[k]], rs, sem).wait()
        pltpu.async_copy(zm.at[didx.at[k]], rd, sem).wait()

        def grp(g, _):
            dv = jnp.zeros((16,), jnp.float32)
            for q in range(16):
                p = g * 16 + q
                a = rs[p, pl.ds(0, 16)] * rd[p, pl.ds(0, 16)]
                for j in range(1, D // 16):
                    a = a + rs[p, pl.ds(j * 16, 16)] * rd[p, pl.ds(j * 16, 16)]
                for sh in (8, 4, 2, 1):
                    a = a + a[lane ^ sh]
                dv = jnp.where(lane == q, a, dv)
            obuf[pl.ds(g * 16, 16)] = dv
            return 0

        lax.fori_loop(0, C // 16, grp, 0)
        pltpu.sync_copy(obuf, out.at[pl.ds(base + k * C, C)])
        return 0

    lax.fori_loop(0, LCHUNK, _chunk, 0)


_decoder = pl.kernel(
    _dec_body,
    out_type=jax.ShapeDtypeStruct((NW * LPT,), jnp.float32),
    mesh=_mesh,
    scratch_types=[
        pltpu.VMEM((LCHUNK, C), jnp.int32),
        pltpu.VMEM((LCHUNK, C), jnp.int32),
        pltpu.VMEM((C, D), jnp.float32),
        pltpu.VMEM((C, D), jnp.float32),
        pltpu.VMEM((C,), jnp.float32),
        pltpu.SemaphoreType.DMA,
    ],
)


_RB = 1024
_GRID = NPAD // _RB


def _mm_body(x, w, o):
    o[...] = jnp.dot(x[...], w[...], preferred_element_type=jnp.float32)


def _mm(x, w):
    return pl.pallas_call(
        _mm_body,
        grid=(_GRID,),
        in_specs=[
            pl.BlockSpec((_RB, D), lambda i: (i, 0)),
            pl.BlockSpec((D, D), lambda i: (0, 0)),
        ],
        out_specs=pl.BlockSpec((_RB, D), lambda i: (i, 0)),
        out_shape=jax.ShapeDtypeStruct((NPAD, D), jnp.float32),
    )(x, w)


def _fin1_body(pacc, pcnt, x, wroot, wnext, b, h, t):
    acc = pacc[0] + pacc[1]
    cnt = pcnt[0] + pcnt[1]
    inv = 1.0 / jnp.maximum(cnt, 1.0)
    hv = jnp.maximum(
        acc * inv[:, None]
        + jnp.dot(x[...], wroot[...], preferred_element_type=jnp.float32)
        + b[...], 0.0)
    h[...] = hv
    t[...] = jnp.dot(hv, wnext[...], preferred_element_type=jnp.float32)


def _finish1(pacc, pcnt, x, wroot, b, wnext):
    return pl.pallas_call(
        _fin1_body,
        grid=(_GRID,),
        in_specs=[
            pl.BlockSpec((NC, _RB, D), lambda i: (0, i, 0)),
            pl.BlockSpec((NC, _RB), lambda i: (0, i)),
            pl.BlockSpec((_RB, D), lambda i: (i, 0)),
            pl.BlockSpec((D, D), lambda i: (0, 0)),
            pl.BlockSpec((D, D), lambda i: (0, 0)),
            pl.BlockSpec((1, D), lambda i: (0, 0)),
        ],
        out_specs=[
            pl.BlockSpec((_RB, D), lambda i: (i, 0)),
            pl.BlockSpec((_RB, D), lambda i: (i, 0)),
        ],
        out_shape=[
            jax.ShapeDtypeStruct((NPAD, D), jnp.float32),
            jax.ShapeDtypeStruct((NPAD, D), jnp.float32),
        ],
    )(pacc, pcnt, x, wroot, wnext, b.reshape(1, D))


def _fin2_body(pacc, pcnt, x, wroot, b, z):
    acc = pacc[0] + pacc[1]
    cnt = pcnt[0] + pcnt[1]
    inv = 1.0 / jnp.maximum(cnt, 1.0)
    z[...] = (acc * inv[:, None]
              + jnp.dot(x[...], wroot[...], preferred_element_type=jnp.float32)
              + b[...])


def _finish2(pacc, pcnt, x, wroot, b):
    return pl.pallas_call(
        _fin2_body,
        grid=(_GRID,),
        in_specs=[
            pl.BlockSpec((NC, _RB, D), lambda i: (0, i, 0)),
            pl.BlockSpec((NC, _RB), lambda i: (0, i)),
            pl.BlockSpec((_RB, D), lambda i: (i, 0)),
            pl.BlockSpec((D, D), lambda i: (0, 0)),
            pl.BlockSpec((1, D), lambda i: (0, 0)),
        ],
        out_specs=pl.BlockSpec((_RB, D), lambda i: (i, 0)),
        out_shape=jax.ShapeDtypeStruct((NPAD, D), jnp.float32),
    )(pacc, pcnt, x, wroot, b.reshape(1, D))


def kernel(x_user, x_movie,
           W1_um_r, W1_um_root, b1_m, W1_mu_r, W1_mu_root, b1_u,
           W2_um_r, W2_um_root, b2_m, W2_mu_r, W2_mu_root, b2_u,
           edge_index_user_movie, edge_index_movie_user, edge_label_index):
    eum = edge_index_user_movie.astype(jnp.int32)
    emu = edge_index_movie_user.astype(jnp.int32)
    eli = edge_label_index.astype(jnp.int32)
    xp_user = jnp.pad(x_user, ((0, NPAD - N), (0, 0)))
    xp_movie = jnp.pad(x_movie, ((0, NPAD - N), (0, 0)))

    epad = EPT - E // NW
    pad_src = (jnp.arange(epad, dtype=jnp.int32) % 16)
    pad_dst = N + (jnp.arange(epad, dtype=jnp.int32) % (NPAD - N))
    src_um = _pad_tiles(eum[0], E // NW, epad, pad_src)
    dst_um = _pad_tiles(eum[1], E // NW, epad, pad_dst)
    src_mu = _pad_tiles(emu[0], E // NW, epad, pad_src)
    dst_mu = _pad_tiles(emu[1], E // NW, epad, pad_dst)

    lpad = LPT - LREAL
    pad_l = (jnp.arange(lpad, dtype=jnp.int32) % 16)
    src_l = _pad_tiles(eli[0], LREAL, lpad, pad_l).reshape(NW, LCHUNK, C)
    dst_l = _pad_tiles(eli[1], LREAL, lpad, pad_l).reshape(NW, LCHUNK, C)

    t_u1 = _mm(xp_user, W1_um_r)
    t_m1 = _mm(xp_movie, W1_mu_r)
    pacc_m, pacc_u, pcnt_m, pcnt_u = _agg2_cnt(
        t_u1, t_m1, src_um, dst_um, src_mu, dst_mu)
    h_m, t_m2 = _finish1(pacc_m, pcnt_m, xp_movie, W1_um_root, b1_m, W2_mu_r)
    h_u, t_u2 = _finish1(pacc_u, pcnt_u, xp_user, W1_mu_root, b1_u, W2_um_r)

    pacc_m2, pacc_u2 = _agg2(t_u2, t_m2, src_um, dst_um, src_mu, dst_mu)
    z_m = _finish2(pacc_m2, pcnt_m, h_m, W2_um_root, b2_m)
    z_u = _finish2(pacc_u2, pcnt_u, h_u, W2_mu_root, b2_u)

    out_pad = _decoder(z_u, z_m, src_l, dst_l)
    return out_pad.reshape(NW, LPT)[:, :LREAL].reshape(-1)


# speedup vs baseline: 1.4931x; 1.0676x over previous
import jax
import jax.numpy as jnp
from jax import lax
from jax.experimental import pallas as pl
from jax.experimental.pallas import tpu as pltpu
from jax.experimental.pallas import tpu_sc as plsc

N = 10000
D = 128
E = 320000
L = 100000

NC, NS = 2, 16
NW = NC * NS
C = 128
NCHUNK = 80
EPT = NCHUNK * C
NPAD = 10240
RPS = NPAD // NS
LCHUNK = 26
LPT = LCHUNK * C
LREAL = L // NW

_mesh = plsc.VectorSubcoreMesh(core_axis_name="c", subcore_axis_name="s")


def _pad_tiles(arr, per_tile, pad_per_tile, pad_vals):
    a = arr.reshape(NW, per_tile)
    pad = jnp.broadcast_to(pad_vals[None, :], (NW, pad_per_tile))
    return jnp.concatenate([a, pad], axis=1).reshape(-1)


def _zero16():
    return jnp.zeros((16,), jnp.float32)


def _agg2_body(with_count, xtA, xtB, srcA, dstA, srcB, dstB, *rest):
    if with_count:
        (outA, outB, cntA, cntB, six0, dix0, six1, dix1, rb0, rb1,
         zbuf, ones, zcnt, accS, cntS, g0, g1, s0, s1, c0, c1) = rest
    else:
        (outA, outB, six0, dix0, six1, dix1, rb0, rb1,
         zbuf, ones, zcnt, accS, cntS, g0, g1, s0, s1, c0, c1) = rest
        cntA = cntB = None
    cid = lax.axis_index("c")
    sid = lax.axis_index("s")
    wid = cid * NS + sid

    z16 = _zero16()
    for i in range(64):
        for j in range(D // 16):
            zbuf[i, pl.ds(j * 16, 16)] = z16
    if with_count:
        one16 = jnp.ones((16,), jnp.float32)
        for j in range(C // 16):
            ones[pl.ds(j * 16, 16)] = one16
        for j in range(RPS // 16):
            zcnt[pl.ds(j * 16, 16)] = z16

    def one_conv(xt, srcF, dstF, out, cnt_out):
        ebase = wid * EPT
        for t in range(RPS // 64):
            pltpu.sync_copy(zbuf, accS.at[pl.ds(sid * RPS + t * 64, 64)])
        if with_count:
            pltpu.sync_copy(zcnt, cntS.at[pl.ds(sid * RPS, RPS)])
        plsc.subcore_barrier()

        six = (six0, six1)
        dix = (dix0, dix1)
        rb = (rb0, rb1)
        gs = (g0, g1)
        ss = (s0, s1)
        cs = (c0, c1)

        def idx_copy(k, p):
            b = pl.multiple_of(ebase + k * C, 8)
            pltpu.sync_copy(srcF.at[pl.ds(b, C)], six[p])
            pltpu.sync_copy(dstF.at[pl.ds(b, C)], dix[p])

        def g_start(k, p):
            pltpu.async_copy(xt.at[six[p]], rb[p], gs[p])

        def g_wait(p):
            pltpu.make_async_copy(xt.at[six[p]], rb[p], gs[p]).wait()

        def s_start(k, p):
            pltpu.async_copy(rb[p], accS.at[dix[p]], ss[p], add=True)
            if with_count:
                pltpu.async_copy(ones, cntS.at[dix[p]], cs[p], add=True)

        def s_wait(p):
            pltpu.make_async_copy(rb[p], accS.at[dix[p]], ss[p]).wait()
            if with_count:
                pltpu.make_async_copy(ones, cntS.at[dix[p]], cs[p]).wait()

        idx_copy(0, 0)
        g_start(0, 0)
        idx_copy(1, 1)
        g_start(1, 1)

        def it(i, _):
            k0 = 2 * i
            g_wait(0)
            s_start(k0, 0)
            g_wait(1)
            s_start(k0 + 1, 1)
            s_wait(0)
            idx_copy(k0 + 2, 0)
            g_start(k0 + 2, 0)
            s_wait(1)
            idx_copy(k0 + 3, 1)
            g_start(k0 + 3, 1)
            return 0

        lax.fori_loop(0, NCHUNK // 2 - 1, it, 0)
        g_wait(0)
        s_start(NCHUNK - 2, 0)
        g_wait(1)
        s_start(NCHUNK - 1, 1)
        s_wait(0)
        s_wait(1)

        plsc.subcore_barrier()
        rr = sid * RPS
        pltpu.sync_copy(accS.at[pl.ds(rr, RPS)], out.at[cid, pl.ds(rr, RPS)])
        if with_count:
            pltpu.sync_copy(cntS.at[pl.ds(rr, RPS)],
                            cnt_out.at[cid, pl.ds(rr, RPS)])
        plsc.subcore_barrier()

    one_conv(xtA, srcA, dstA, outA, cntA)
    one_conv(xtB, srcB, dstB, outB, cntB)


import functools as _ft


def _make_agg2(with_count):
    outs = [jax.ShapeDtypeStruct((NC, NPAD, D), jnp.float32),
            jax.ShapeDtypeStruct((NC, NPAD, D), jnp.float32)]
    if with_count:
        outs += [jax.ShapeDtypeStruct((NC, NPAD), jnp.float32),
                 jax.ShapeDtypeStruct((NC, NPAD), jnp.float32)]
    return pl.kernel(
        _ft.partial(_agg2_body, with_count),
        out_type=tuple(outs),
        mesh=_mesh,
        scratch_types=[
            pltpu.VMEM((C,), jnp.int32),
            pltpu.VMEM((C,), jnp.int32),
            pltpu.VMEM((C,), jnp.int32),
            pltpu.VMEM((C,), jnp.int32),
            pltpu.VMEM((C, D), jnp.float32),
            pltpu.VMEM((C, D), jnp.float32),
            pltpu.VMEM((64, D), jnp.float32),
            pltpu.VMEM((C,), jnp.float32),
            pltpu.VMEM((RPS,), jnp.float32),
            pltpu.VMEM_SHARED((NPAD, D), jnp.float32),
            pltpu.VMEM_SHARED((NPAD,), jnp.float32),
            pltpu.SemaphoreType.DMA,
            pltpu.SemaphoreType.DMA,
            pltpu.SemaphoreType.DMA,
            pltpu.SemaphoreType.DMA,
            pltpu.SemaphoreType.DMA,
            pltpu.SemaphoreType.DMA,
        ],
    )


_agg2_cnt = _make_agg2(True)
_agg2 = _make_agg2(False)


def _dec_body(zu, zm, srcA, dstA, out,
              sidx, didx, rs0, rd0, rs1, rd1, ob0, ob1,
              grs0, grd0, grs1, grd1, os0, os1):
    cid = lax.axis_index("c")
    sid = lax.axis_index("s")
    wid = cid * NS + sid
    base = wid * LPT
    pltpu.sync_copy(srcA.at[wid], sidx)
    pltpu.sync_copy(dstA.at[wid], didx)
    lane = lax.iota(jnp.int32, 16)
    rs = (rs0, rs1)
    rd = (rd0, rd1)
    ob = (ob0, ob1)
    grs = (grs0, grs1)
    grd = (grd0, grd1)
    os_ = (os0, os1)

    def g_start(k, p):
        pltpu.async_copy(zu.at[sidx.at[k]], rs[p], grs[p])
        pltpu.async_copy(zm.at[didx.at[k]], rd[p], grd[p])

    def g_wait(p):
        pltpu.make_async_copy(zu.at[sidx.at[0]], rs[p], grs[p]).wait()
        pltpu.make_async_copy(zm.at[didx.at[0]], rd[p], grd[p]).wait()

    def o_start(k, p):
        pltpu.async_copy(ob[p], out.at[pl.ds(base + k * C, C)], os_[p])

    def o_wait(p):
        pltpu.make_async_copy(ob[p], out.at[pl.ds(base, C)], os_[p]).wait()

    def compute(p):
        def grp(g, _):
            dv = jnp.zeros((16,), jnp.float32)
            for q in range(16):
                pr = g * 16 + q
                a = rs[p][pr, pl.ds(0, 16)] * rd[p][pr, pl.ds(0, 16)]
                for j in range(1, D // 16):
                    a = a + (rs[p][pr, pl.ds(j * 16, 16)]
                             * rd[p][pr, pl.ds(j * 16, 16)])
                for sh in (8, 4, 2, 1):
                    a = a + a[lane ^ sh]
                dv = jnp.where(lane == q, a, dv)
            ob[p][pl.ds(g * 16, 16)] = dv
            return 0
        lax.fori_loop(0, C // 16, grp, 0)

    g_start(0, 0)
    g_start(1, 1)
    g_wait(0)
    compute(0)
    o_start(0, 0)
    g_start(2, 0)
    g_wait(1)
    compute(1)
    o_start(1, 1)
    g_start(3, 1)

    def it(i, _):
        k0 = 2 * i
        g_wait(0)
        o_wait(0)
        compute(0)
        o_start(k0, 0)
        g_start(k0 + 2, 0)
        g_wait(1)
        o_wait(1)
        compute(1)
        o_start(k0 + 1, 1)
        g_start(k0 + 3, 1)
        return 0

    lax.fori_loop(1, LCHUNK // 2 - 1, it, 0)
    g_wait(0)
    o_wait(0)
    compute(0)
    o_start(LCHUNK - 2, 0)
    g_wait(1)
    o_wait(1)
    compute(1)
    o_start(LCHUNK - 1, 1)
    o_wait(0)
    o_wait(1)


_decoder = pl.kernel(
    _dec_body,
    out_type=jax.ShapeDtypeStruct((NW * LPT,), jnp.float32),
    mesh=_mesh,
    scratch_types=[
        pltpu.VMEM((LCHUNK, C), jnp.int32),
        pltpu.VMEM((LCHUNK, C), jnp.int32),
        pltpu.VMEM((C, D), jnp.float32),
        pltpu.VMEM((C, D), jnp.float32),
        pltpu.VMEM((C, D), jnp.float32),
        pltpu.VMEM((C, D), jnp.float32),
        pltpu.VMEM((C,), jnp.float32),
        pltpu.VMEM((C,), jnp.float32),
        pltpu.SemaphoreType.DMA,
        pltpu.SemaphoreType.DMA,
        pltpu.SemaphoreType.DMA,
        pltpu.SemaphoreType.DMA,
        pltpu.SemaphoreType.DMA,
        pltpu.SemaphoreType.DMA,
    ],
)


_RB = 1024
_GRID = NPAD // _RB


def _mm_body(x, w, o):
    o[...] = jnp.dot(x[...], w[...], preferred_element_type=jnp.float32)


def _mm(x, w):
    return pl.pallas_call(
        _mm_body,
        grid=(_GRID,),
        in_specs=[
            pl.BlockSpec((_RB, D), lambda i: (i, 0)),
            pl.BlockSpec((D, D), lambda i: (0, 0)),
        ],
        out_specs=pl.BlockSpec((_RB, D), lambda i: (i, 0)),
        out_shape=jax.ShapeDtypeStruct((NPAD, D), jnp.float32),
    )(x, w)


def _fin1_body(pacc, pcnt, x, wroot, wnext, b, h, t):
    acc = pacc[0] + pacc[1]
    cnt = pcnt[0] + pcnt[1]
    inv = 1.0 / jnp.maximum(cnt, 1.0)
    hv = jnp.maximum(
        acc * inv[:, None]
        + jnp.dot(x[...], wroot[...], preferred_element_type=jnp.float32)
        + b[...], 0.0)
    h[...] = hv
    t[...] = jnp.dot(hv, wnext[...], preferred_element_type=jnp.float32)


def _finish1(pacc, pcnt, x, wroot, b, wnext):
    return pl.pallas_call(
        _fin1_body,
        grid=(_GRID,),
        in_specs=[
            pl.BlockSpec((NC, _RB, D), lambda i: (0, i, 0)),
            pl.BlockSpec((NC, _RB), lambda i: (0, i)),
            pl.BlockSpec((_RB, D), lambda i: (i, 0)),
            pl.BlockSpec((D, D), lambda i: (0, 0)),
            pl.BlockSpec((D, D), lambda i: (0, 0)),
            pl.BlockSpec((1, D), lambda i: (0, 0)),
        ],
        out_specs=[
            pl.BlockSpec((_RB, D), lambda i: (i, 0)),
            pl.BlockSpec((_RB, D), lambda i: (i, 0)),
        ],
        out_shape=[
            jax.ShapeDtypeStruct((NPAD, D), jnp.float32),
            jax.ShapeDtypeStruct((NPAD, D), jnp.float32),
        ],
    )(pacc, pcnt, x, wroot, wnext, b.reshape(1, D))


def _fin2_body(pacc, pcnt, x, wroot, b, z):
    acc = pacc[0] + pacc[1]
    cnt = pcnt[0] + pcnt[1]
    inv = 1.0 / jnp.maximum(cnt, 1.0)
    z[...] = (acc * inv[:, None]
              + jnp.dot(x[...], wroot[...], preferred_element_type=jnp.float32)
              + b[...])


def _finish2(pacc, pcnt, x, wroot, b):
    return pl.pallas_call(
        _fin2_body,
        grid=(_GRID,),
        in_specs=[
            pl.BlockSpec((NC, _RB, D), lambda i: (0, i, 0)),
            pl.BlockSpec((NC, _RB), lambda i: (0, i)),
            pl.BlockSpec((_RB, D), lambda i: (i, 0)),
            pl.BlockSpec((D, D), lambda i: (0, 0)),
            pl.BlockSpec((1, D), lambda i: (0, 0)),
        ],
        out_specs=pl.BlockSpec((_RB, D), lambda i: (i, 0)),
        out_shape=jax.ShapeDtypeStruct((NPAD, D), jnp.float32),
    )(pacc, pcnt, x, wroot, b.reshape(1, D))


def kernel(x_user, x_movie,
           W1_um_r, W1_um_root, b1_m, W1_mu_r, W1_mu_root, b1_u,
           W2_um_r, W2_um_root, b2_m, W2_mu_r, W2_mu_root, b2_u,
           edge_index_user_movie, edge_index_movie_user, edge_label_index):
    eum = edge_index_user_movie.astype(jnp.int32)
    emu = edge_index_movie_user.astype(jnp.int32)
    eli = edge_label_index.astype(jnp.int32)
    xp_user = jnp.pad(x_user, ((0, NPAD - N), (0, 0)))
    xp_movie = jnp.pad(x_movie, ((0, NPAD - N), (0, 0)))

    epad = EPT - E // NW
    pad_src = (jnp.arange(epad, dtype=jnp.int32) % 16)
    pad_dst = N + (jnp.arange(epad, dtype=jnp.int32) % (NPAD - N))
    src_um = _pad_tiles(eum[0], E // NW, epad, pad_src)
    dst_um = _pad_tiles(eum[1], E // NW, epad, pad_dst)
    src_mu = _pad_tiles(emu[0], E // NW, epad, pad_src)
    dst_mu = _pad_tiles(emu[1], E // NW, epad, pad_dst)

    lpad = LPT - LREAL
    pad_l = (jnp.arange(lpad, dtype=jnp.int32) % 16)
    src_l = _pad_tiles(eli[0], LREAL, lpad, pad_l).reshape(NW, LCHUNK, C)
    dst_l = _pad_tiles(eli[1], LREAL, lpad, pad_l).reshape(NW, LCHUNK, C)

    t_u1 = _mm(xp_user, W1_um_r)
    t_m1 = _mm(xp_movie, W1_mu_r)
    pacc_m, pacc_u, pcnt_m, pcnt_u = _agg2_cnt(
        t_u1, t_m1, src_um, dst_um, src_mu, dst_mu)
    h_m, t_m2 = _finish1(pacc_m, pcnt_m, xp_movie, W1_um_root, b1_m, W2_mu_r)
    h_u, t_u2 = _finish1(pacc_u, pcnt_u, xp_user, W1_mu_root, b1_u, W2_um_r)

    pacc_m2, pacc_u2 = _agg2(t_u2, t_m2, src_um, dst_um, src_mu, dst_mu)
    z_m = _finish2(pacc_m2, pcnt_m, h_m, W2_um_root, b2_m)
    z_u = _finish2(pacc_u2, pcnt_u, h_u, W2_mu_root, b2_u)

    out_pad = _decoder(z_u, z_m, src_l, dst_l)
    return out_pad.reshape(NW, LPT)[:, :LREAL].reshape(-1)
